# baseline (speedup 1.0000x reference)
import functools

import jax
import jax.numpy as jnp
from jax import lax
from jax.experimental import pallas as pl
from jax.experimental.pallas import tpu as pltpu
from jax.experimental.pallas import tpu_sc as plsc

N = 10000
D = 128
E = 320000
L = 2

NC = 2
NS = 16
NW = NC * NS

CW = 64
CH = 160
HCH = 80
EP = NW * CH * CW
EW = CH * CW

NP = 10240
NPW = NP // NW
NSTRIPE = NP // NS

_mesh = plsc.VectorSubcoreMesh(core_axis_name="c", subcore_axis_name="s")


_EBLK = 2048


def _embed_body(nf_ref, tab_ref, out_ref):
    onehot = (nf_ref[:] == lax.broadcasted_iota(
        jnp.int32, (_EBLK, 128), 1)).astype(jnp.float32)
    out_ref[:] = jnp.dot(onehot, tab_ref[:],
                         preferred_element_type=jnp.float32)


def _embed_tc(nf2d, tab):
    return pl.pallas_call(
        _embed_body,
        grid=(NP // _EBLK,),
        in_specs=[
            pl.BlockSpec((_EBLK, 1), lambda j: (j, 0)),
            pl.BlockSpec((128, D), lambda j: (0, 0)),
        ],
        out_specs=pl.BlockSpec((_EBLK, D), lambda j: (j, 0)),
        out_shape=jax.ShapeDtypeStruct((NP, D), jnp.float32),
    )(nf2d, tab)


def _make_segsum():
    @functools.partial(
        pl.kernel,
        mesh=_mesh,
        out_type=jax.ShapeDtypeStruct((NC * NP, D), jnp.float32),
        scratch_types=[
            pltpu.VMEM((HCH, CW), jnp.int32),
            pltpu.VMEM((HCH, CW), jnp.int32),
            pltpu.VMEM((3, CW, D), jnp.float32),
            pltpu.VMEM_SHARED((NP, D), jnp.float32),
            pltpu.SemaphoreType.DMA,
            pltpu.SemaphoreType.DMA,
            pltpu.SemaphoreType.DMA,
            pltpu.SemaphoreType.DMA,
            pltpu.SemaphoreType.DMA,
            pltpu.SemaphoreType.DMA,
        ],
    )
    def _segsum_sc(h_hbm, src_hbm, dst_hbm, zer_hbm, out_hbm,
                   src_v, dst_v, rows_v, agg_sh,
                   g0, g1, g2, s0, s1, s2):
        c = lax.axis_index("c")
        s = lax.axis_index("s")
        wid = s * NC + c
        gsem = (g0, g1, g2)
        ssem = (s0, s1, s2)
        pltpu.sync_copy(zer_hbm, agg_sh.at[pl.ds(s * NSTRIPE, NSTRIPE)])
        plsc.subcore_barrier()

        def _gather(j, b):
            pltpu.async_copy(h_hbm.at[src_v.at[j]], rows_v.at[b], gsem[b])

        def _wait_gather(j, b):
            pltpu.make_async_copy(
                h_hbm.at[src_v.at[j]], rows_v.at[b], gsem[b]).wait()

        def _scatter(j, b):
            pltpu.async_copy(rows_v.at[b], agg_sh.at[dst_v.at[j]], ssem[b],
                             add=True)

        def _wait_scatter(b):
            pltpu.make_async_copy(rows_v.at[b], agg_sh.at[dst_v.at[0]],
                                  ssem[b]).wait()

        for hh in range(CH // HCH):
            pltpu.sync_copy(src_hbm.at[wid, pl.ds(hh * HCH, HCH)], src_v)
            pltpu.sync_copy(dst_hbm.at[wid, pl.ds(hh * HCH, HCH)], dst_v)
            for b in range(2):
                _gather(b, b)

            def body(g, _):
                for b in range(3):
                    j = 3 * g + b
                    bp = (b + 2) % 3

                    @pl.when(j + 2 < HCH)
                    def _():
                        if b == 0:
                            @pl.when(g >= 1)
                            def _():
                                _wait_scatter(bp)
                        else:
                            _wait_scatter(bp)
                        _gather(j + 2, bp)

                    _wait_gather(j, b)
                    _scatter(j, b)
                return ()

            lax.fori_loop(0, HCH // 3, body, (), unroll=False)
            for t in range(HCH - HCH % 3, HCH):
                b = t % 3
                _wait_gather(t, b)
                _scatter(t, b)
            for b in range(3):
                _wait_scatter(b)
        plsc.subcore_barrier()
        pltpu.sync_copy(agg_sh.at[pl.ds(s * NSTRIPE, NSTRIPE)],
                        out_hbm.at[pl.ds(c * NP + s * NSTRIPE, NSTRIPE)])

    return _segsum_sc


_segsum = _make_segsum()


_BLK = 2000
_GRID = N // _BLK


def _conv1_body(seg_ref, h_ref, wn_ref, ws_ref, b_ref, out_ref, sums_ref):
    j = pl.program_id(0)
    agg = seg_ref[0] + seg_ref[1]
    y = jnp.dot(agg, wn_ref[:], preferred_element_type=jnp.float32)
    y = y + jnp.dot(h_ref[:], ws_ref[:], preferred_element_type=jnp.float32)
    y = y + b_ref[:]
    y = jnp.where(y >= 0, y, 0.2 * y)
    out_ref[:] = y
    part = jnp.concatenate(
        [jnp.sum(h_ref[:], axis=0, keepdims=True),
         jnp.sum(y, axis=0, keepdims=True)], axis=0)

    @pl.when(j == 0)
    def _():
        sums_ref[:] = part

    @pl.when(j > 0)
    def _():
        sums_ref[:] = sums_ref[:] + part


def _conv1(seg, h, wn, ws, b):
    return pl.pallas_call(
        _conv1_body,
        grid=(_GRID,),
        in_specs=[
            pl.BlockSpec((2, _BLK, D), lambda j: (0, j, 0)),
            pl.BlockSpec((_BLK, D), lambda j: (j, 0)),
            pl.BlockSpec((D, D), lambda j: (0, 0)),
            pl.BlockSpec((D, D), lambda j: (0, 0)),
            pl.BlockSpec((1, D), lambda j: (0, 0)),
        ],
        out_specs=[
            pl.BlockSpec((_BLK, D), lambda j: (j, 0)),
            pl.BlockSpec((2, D), lambda j: (0, 0)),
        ],
        out_shape=[
            jax.ShapeDtypeStruct((NP, D), jnp.float32),
            jax.ShapeDtypeStruct((2, D), jnp.float32),
        ],
    )(seg, h, wn, ws, b)


def _conv2_body(seg_ref, h_ref, wn_ref, ws_ref, b_ref, rw_ref, rb_ref,
                sums_ref, out_ref):
    j = pl.program_id(0)
    agg = seg_ref[0] + seg_ref[1]
    y = jnp.dot(agg, wn_ref[:], preferred_element_type=jnp.float32)
    y = y + jnp.dot(h_ref[:], ws_ref[:], preferred_element_type=jnp.float32)
    y = y + b_ref[:]
    y = jnp.where(y >= 0, y, 0.2 * y)
    s2 = jnp.sum(y, axis=0, keepdims=True)
    part = jnp.dot(s2, rw_ref[2], preferred_element_type=jnp.float32)

    @pl.when(j == 0)
    def _():
        base = (jnp.dot(sums_ref[0:1], rw_ref[0],
                        preferred_element_type=jnp.float32)
                + jnp.dot(sums_ref[1:2], rw_ref[1],
                          preferred_element_type=jnp.float32)
                + N * jnp.sum(rb_ref[:], axis=0, keepdims=True))
        out_ref[:] = base + part

    @pl.when(j > 0)
    def _():
        out_ref[:] = out_ref[:] + part


def _conv2(seg, h, wn, ws, b, rw, rb, sums):
    return pl.pallas_call(
        _conv2_body,
        grid=(_GRID,),
        in_specs=[
            pl.BlockSpec((2, _BLK, D), lambda j: (0, j, 0)),
            pl.BlockSpec((_BLK, D), lambda j: (j, 0)),
            pl.BlockSpec((D, D), lambda j: (0, 0)),
            pl.BlockSpec((D, D), lambda j: (0, 0)),
            pl.BlockSpec((1, D), lambda j: (0, 0)),
            pl.BlockSpec((L + 1, D, D), lambda j: (0, 0, 0)),
            pl.BlockSpec((L + 1, D), lambda j: (0, 0)),
            pl.BlockSpec((2, D), lambda j: (0, 0)),
        ],
        out_specs=pl.BlockSpec((1, D), lambda j: (0, 0)),
        out_shape=jax.ShapeDtypeStruct((1, D), jnp.float32),
    )(seg, h, wn, ws, b, rw, rb, sums)


def kernel(nfeats, edge_index, efeats, W_embed, W_neigh, W_self, b_conv, R_w, R_b):
    src = edge_index[0]
    dst = edge_index[1]
    srcp = jnp.concatenate(
        [src, jnp.arange(EP - E, dtype=jnp.int32) % N]).reshape(NW, CH, CW)
    dstp = jnp.concatenate(
        [dst, N + jnp.arange(EP - E, dtype=jnp.int32) % (NP - N)]
    ).reshape(NW, CH, CW)
    nfp = jnp.concatenate(
        [nfeats, jnp.arange(NP - N, dtype=jnp.int32) % 128]).reshape(NP, 1)
    zer = jnp.zeros((NSTRIPE, D), jnp.float32)

    h0 = _embed_tc(nfp, W_embed)
    seg1 = _segsum(h0, srcp, dstp, zer).reshape(NC, NP, D)
    h1, sums01 = _conv1(seg1, h0, W_neigh[0], W_self[0],
                        b_conv[0].reshape(1, D))
    seg2 = _segsum(h1, srcp, dstp, zer).reshape(NC, NP, D)
    return _conv2(seg2, h1, W_neigh[1], W_self[1], b_conv[1].reshape(1, D),
                  R_w, R_b, sums01)


# speedup vs baseline: 1.0268x; 1.0268x over previous
import functools

import jax
import jax.numpy as jnp
from jax import lax
from jax.experimental import pallas as pl
from jax.experimental.pallas import tpu as pltpu
from jax.experimental.pallas import tpu_sc as plsc

N = 10000
D = 128
E = 320000
L = 2

NC = 2
NS = 16
NW = NC * NS

CW = 64
CH = 160
HCH = 80
EP = NW * CH * CW
EW = CH * CW

NP = 10240
NPW = NP // NW
NSTRIPE = NP // NS

_mesh = plsc.VectorSubcoreMesh(core_axis_name="c", subcore_axis_name="s")


@functools.partial(
    pl.kernel,
    mesh=_mesh,
    out_type=jax.ShapeDtypeStruct((NP, D), jnp.float32),
    scratch_types=[
        pltpu.VMEM((4, 80), jnp.int32),
        pltpu.VMEM((2, 80, D), jnp.float32),
        pltpu.SemaphoreType.DMA,
        pltpu.SemaphoreType.DMA,
    ],
)
def _embed_sc(tab_hbm, idx_hbm, out_hbm, idx_v, rows_v, sem0, sem1):
    c = lax.axis_index("c")
    s = lax.axis_index("s")
    wid = s * NC + c
    sems = (sem0, sem1)
    pltpu.sync_copy(idx_hbm.at[wid], idx_v)

    pltpu.async_copy(tab_hbm.at[idx_v.at[0]], rows_v.at[0], sems[0])
    for j in range(4):
        b = j % 2
        if j + 1 < 4:
            pltpu.async_copy(
                tab_hbm.at[idx_v.at[j + 1]], rows_v.at[1 - b], sems[1 - b])
        pltpu.make_async_copy(
            tab_hbm.at[idx_v.at[j]], rows_v.at[b], sems[b]).wait()
        pltpu.sync_copy(rows_v.at[b],
                        out_hbm.at[pl.ds(wid * NPW + j * 80, 80)])


def _make_segsum():
    @functools.partial(
        pl.kernel,
        mesh=_mesh,
        out_type=jax.ShapeDtypeStruct((NC * NP, D), jnp.float32),
        scratch_types=[
            pltpu.VMEM((HCH, CW), jnp.int32),
            pltpu.VMEM((HCH, CW), jnp.int32),
            pltpu.VMEM((3, CW, D), jnp.float32),
            pltpu.VMEM_SHARED((NP, D), jnp.float32),
            pltpu.SemaphoreType.DMA,
            pltpu.SemaphoreType.DMA,
            pltpu.SemaphoreType.DMA,
            pltpu.SemaphoreType.DMA,
            pltpu.SemaphoreType.DMA,
            pltpu.SemaphoreType.DMA,
        ],
    )
    def _segsum_sc(h_hbm, src_hbm, dst_hbm, zer_hbm, out_hbm,
                   src_v, dst_v, rows_v, agg_sh,
                   g0, g1, g2, s0, s1, s2):
        c = lax.axis_index("c")
        s = lax.axis_index("s")
        wid = s * NC + c
        gsem = (g0, g1, g2)
        ssem = (s0, s1, s2)
        pltpu.sync_copy(zer_hbm, agg_sh.at[pl.ds(s * NSTRIPE, NSTRIPE)])
        plsc.subcore_barrier()

        def _gather(j, b):
            pltpu.async_copy(h_hbm.at[src_v.at[j]], rows_v.at[b], gsem[b])

        def _wait_gather(j, b):
            pltpu.make_async_copy(
                h_hbm.at[src_v.at[j]], rows_v.at[b], gsem[b]).wait()

        def _scatter(j, b):
            pltpu.async_copy(rows_v.at[b], agg_sh.at[dst_v.at[j]], ssem[b],
                             add=True)

        def _wait_scatter(b):
            pltpu.make_async_copy(rows_v.at[b], agg_sh.at[dst_v.at[0]],
                                  ssem[b]).wait()

        for hh in range(CH // HCH):
            pltpu.sync_copy(src_hbm.at[wid, pl.ds(hh * HCH, HCH)], src_v)
            pltpu.sync_copy(dst_hbm.at[wid, pl.ds(hh * HCH, HCH)], dst_v)
            for b in range(2):
                _gather(b, b)

            def body(g, _):
                for b in range(3):
                    j = 3 * g + b
                    bp = (b + 2) % 3

                    @pl.when(j + 2 < HCH)
                    def _():
                        if b == 0:
                            @pl.when(g >= 1)
                            def _():
                                _wait_scatter(bp)
                        else:
                            _wait_scatter(bp)
                        _gather(j + 2, bp)

                    _wait_gather(j, b)
                    _scatter(j, b)
                return ()

            lax.fori_loop(0, HCH // 3, body, (), unroll=False)
            for t in range(HCH - HCH % 3, HCH):
                b = t % 3
                _wait_gather(t, b)
                _scatter(t, b)
            for b in range(3):
                _wait_scatter(b)
        plsc.subcore_barrier()
        pltpu.sync_copy(agg_sh.at[pl.ds(s * NSTRIPE, NSTRIPE)],
                        out_hbm.at[pl.ds(c * NP + s * NSTRIPE, NSTRIPE)])

    return _segsum_sc


_segsum = _make_segsum()


_BLK = 2000
_GRID = N // _BLK


def _conv1_body(seg_ref, h_ref, wn_ref, ws_ref, b_ref, out_ref, sums_ref):
    j = pl.program_id(0)
    agg = seg_ref[0] + seg_ref[1]
    y = jnp.dot(agg, wn_ref[:], preferred_element_type=jnp.float32)
    y = y + jnp.dot(h_ref[:], ws_ref[:], preferred_element_type=jnp.float32)
    y = y + b_ref[:]
    y = jnp.where(y >= 0, y, 0.2 * y)
    out_ref[:] = y
    part = jnp.concatenate(
        [jnp.sum(h_ref[:], axis=0, keepdims=True),
         jnp.sum(y, axis=0, keepdims=True)], axis=0)

    @pl.when(j == 0)
    def _():
        sums_ref[:] = part

    @pl.when(j > 0)
    def _():
        sums_ref[:] = sums_ref[:] + part


def _conv1(seg, h, wn, ws, b):
    return pl.pallas_call(
        _conv1_body,
        grid=(_GRID,),
        in_specs=[
            pl.BlockSpec((2, _BLK, D), lambda j: (0, j, 0)),
            pl.BlockSpec((_BLK, D), lambda j: (j, 0)),
            pl.BlockSpec((D, D), lambda j: (0, 0)),
            pl.BlockSpec((D, D), lambda j: (0, 0)),
            pl.BlockSpec((1, D), lambda j: (0, 0)),
        ],
        out_specs=[
            pl.BlockSpec((_BLK, D), lambda j: (j, 0)),
            pl.BlockSpec((2, D), lambda j: (0, 0)),
        ],
        out_shape=[
            jax.ShapeDtypeStruct((NP, D), jnp.float32),
            jax.ShapeDtypeStruct((2, D), jnp.float32),
        ],
    )(seg, h, wn, ws, b)


def _conv2_body(seg_ref, h_ref, wn_ref, ws_ref, b_ref, rw_ref, rb_ref,
                sums_ref, out_ref):
    j = pl.program_id(0)
    agg = seg_ref[0] + seg_ref[1]
    y = jnp.dot(agg, wn_ref[:], preferred_element_type=jnp.float32)
    y = y + jnp.dot(h_ref[:], ws_ref[:], preferred_element_type=jnp.float32)
    y = y + b_ref[:]
    y = jnp.where(y >= 0, y, 0.2 * y)
    s2 = jnp.sum(y, axis=0, keepdims=True)
    part = jnp.dot(s2, rw_ref[2], preferred_element_type=jnp.float32)

    @pl.when(j == 0)
    def _():
        base = (jnp.dot(sums_ref[0:1], rw_ref[0],
                        preferred_element_type=jnp.float32)
                + jnp.dot(sums_ref[1:2], rw_ref[1],
                          preferred_element_type=jnp.float32)
                + N * jnp.sum(rb_ref[:], axis=0, keepdims=True))
        out_ref[:] = base + part

    @pl.when(j > 0)
    def _():
        out_ref[:] = out_ref[:] + part


def _conv2(seg, h, wn, ws, b, rw, rb, sums):
    return pl.pallas_call(
        _conv2_body,
        grid=(_GRID,),
        in_specs=[
            pl.BlockSpec((2, _BLK, D), lambda j: (0, j, 0)),
            pl.BlockSpec((_BLK, D), lambda j: (j, 0)),
            pl.BlockSpec((D, D), lambda j: (0, 0)),
            pl.BlockSpec((D, D), lambda j: (0, 0)),
            pl.BlockSpec((1, D), lambda j: (0, 0)),
            pl.BlockSpec((L + 1, D, D), lambda j: (0, 0, 0)),
            pl.BlockSpec((L + 1, D), lambda j: (0, 0)),
            pl.BlockSpec((2, D), lambda j: (0, 0)),
        ],
        out_specs=pl.BlockSpec((1, D), lambda j: (0, 0)),
        out_shape=jax.ShapeDtypeStruct((1, D), jnp.float32),
    )(seg, h, wn, ws, b, rw, rb, sums)


def kernel(nfeats, edge_index, efeats, W_embed, W_neigh, W_self, b_conv, R_w, R_b):
    src = edge_index[0]
    dst = edge_index[1]
    srcp = jnp.concatenate(
        [src, jnp.arange(EP - E, dtype=jnp.int32) % N]).reshape(NW, CH, CW)
    dstp = jnp.concatenate(
        [dst, N + jnp.arange(EP - E, dtype=jnp.int32) % (NP - N)]
    ).reshape(NW, CH, CW)
    nfp = jnp.concatenate(
        [nfeats, jnp.arange(NP - N, dtype=jnp.int32) % 128]).reshape(NW, 4, 80)
    zer = jnp.zeros((NSTRIPE, D), jnp.float32)

    h0 = _embed_sc(W_embed, nfp)
    seg1 = _segsum(h0, srcp, dstp, zer).reshape(NC, NP, D)
    h1, sums01 = _conv1(seg1, h0, W_neigh[0], W_self[0],
                        b_conv[0].reshape(1, D))
    seg2 = _segsum(h1, srcp, dstp, zer).reshape(NC, NP, D)
    return _conv2(seg2, h1, W_neigh[1], W_self[1], b_conv[1].reshape(1, D),
                  R_w, R_b, sums01)
